# Initial kernel scaffold; baseline (speedup 1.0000x reference)
import jax
import jax.numpy as jnp
from jax.experimental import pallas as pl


def kernel(attn_bias, spatial_pos, x, edge_input, edge_encoder_w, spatial_pos_encoder_w, graph_token_virtual_distance_w, edge_dis_encoder_w):
    raise NotImplementedError("write your pallas kernel here")


# speedup vs baseline: 12.6323x; 12.6323x over previous
import functools

import jax
import jax.numpy as jnp
from jax import lax
from jax.experimental import pallas as pl
from jax.experimental.pallas import tpu as pltpu
from jax.experimental.pallas import tpu_sc as plsc

B, N, H = 16, 64, 32
NUM_EDGES, NUM_SPATIAL, MAX_DIST = 1536, 512, 5
S = B * N * N
VPAD = 1544
NW = 32
NQ = 4
NGRP = NW // NQ
S_PER_W = S // NGRP
CHUNK = 512
L = 16


def _prep_body(e_ref, w5_ref, spt_ref, tab_ref):
    for d in range(MAX_DIST):
        tab_ref[d] = lax.dot_general(
            w5_ref[d], e_ref[...], (((0,), (1,)), ((), ())),
            preferred_element_type=jnp.float32)
    tab_ref[MAX_DIST] = spt_ref[...]


def _make_prep():
    return pl.pallas_call(
        _prep_body,
        out_shape=jax.ShapeDtypeStruct((MAX_DIST + 1, H, VPAD), jnp.float32),
    )


def _sc_body(tab_hbm, eidx_hbm, sp_hbm, core_hbm, tab_v, eidx_v, sp_v, out_v):
    wid = lax.axis_index("s") * 2 + lax.axis_index("c")
    q = wid % NQ
    grp = wid // NQ
    pltpu.sync_copy(tab_hbm.at[q], tab_v)
    iota = lax.iota(jnp.int32, L)

    def chunk_body(ci):
        sbase = grp * S_PER_W + ci * CHUNK
        b = sbase // (N * N)
        i0 = pl.multiple_of((sbase % (N * N)) // N, CHUNK // N)
        pltpu.sync_copy(eidx_hbm.at[pl.ds(sbase * 15, CHUNK * 15)], eidx_v)
        pltpu.sync_copy(sp_hbm.at[pl.ds(sbase, CHUNK)], sp_v)

        def group_body(g):
            rows15 = (iota + g * L) * 15
            sp = sp_v[pl.ds(g * L, L)]
            s4 = jnp.minimum(jnp.maximum(sp - 1, 1), MAX_DIST)
            scale = (1.0 / 3.0) / s4.astype(jnp.float32)
            accs = []
            for c in range(8):
                accs.append(jnp.zeros((L,), jnp.float32))
            for k in range(15):
                d = k // 3
                r = plsc.load_gather(eidx_v, [rows15 + k])
                for c in range(8):
                    accs[c] = accs[c] + plsc.load_gather(
                        tab_v, [r + (d * 8 + c) * VPAD])
            i_loc = g // 4
            j0 = (g % 4) * L
            for c in range(8):
                spb = plsc.load_gather(
                    tab_v, [sp + (MAX_DIST * 8 + c) * VPAD])
                out_v[c, i_loc, pl.ds(j0, L)] = accs[c] * scale + spb

        pl.loop(0, CHUNK // L)(group_body)
        pltpu.sync_copy(
            out_v,
            core_hbm.at[b, pl.ds(q * 8, 8), pl.ds(i0, CHUNK // N), :])

    pl.loop(0, S_PER_W // CHUNK)(chunk_body)


def _make_sc():
    mesh = plsc.VectorSubcoreMesh(core_axis_name="c", subcore_axis_name="s")
    return functools.partial(
        pl.kernel,
        out_type=jax.ShapeDtypeStruct((B, H, N, N), jnp.float32),
        mesh=mesh,
        compiler_params=pltpu.CompilerParams(needs_layout_passes=False),
        scratch_types=[
            pltpu.VMEM(((MAX_DIST + 1) * 8 * VPAD,), jnp.float32),
            pltpu.VMEM((CHUNK * 15,), jnp.int32),
            pltpu.VMEM((CHUNK,), jnp.int32),
            pltpu.VMEM((8, CHUNK // N, N), jnp.float32),
        ],
    )(_sc_body)


def _asm_body(ab_ref, t_ref, core_ref, out_ref):
    cr = core_ref[...][0]
    tt = t_ref[...]
    top = jnp.broadcast_to(tt[:, None, :], (H, 1, N))
    inner = jnp.concatenate([top, cr], axis=1)
    left = jnp.broadcast_to(tt[:, None, :], (H, N + 1, 1))
    x = jnp.concatenate([left, inner], axis=2)
    ab2 = ab_ref[...] * 2.0
    out_ref[...] = (ab2[:, None, :, :] + x[None])


def _make_asm():
    return pl.pallas_call(
        _asm_body,
        grid=(B,),
        in_specs=[
            pl.BlockSpec((1, N + 1, N + 1), lambda b: (b, 0, 0)),
            pl.BlockSpec((H, 1), lambda b: (0, 0)),
            pl.BlockSpec((1, H, N, N), lambda b: (b, 0, 0, 0)),
        ],
        out_specs=pl.BlockSpec((1, H, N + 1, N + 1), lambda b: (b, 0, 0, 0)),
        out_shape=jax.ShapeDtypeStruct((B, H, N + 1, N + 1), jnp.float32),
    )


def kernel(attn_bias, spatial_pos, x, edge_input, edge_encoder_w,
           spatial_pos_encoder_w, graph_token_virtual_distance_w,
           edge_dis_encoder_w):
    del x
    w5 = edge_dis_encoder_w.reshape(-1, H, H)[:MAX_DIST]
    e_pad = jnp.pad(edge_encoder_w, ((0, VPAD - edge_encoder_w.shape[0]),
                                     (0, 0)))
    spt = jnp.pad(spatial_pos_encoder_w,
                  ((0, VPAD - NUM_SPATIAL), (0, 0))).T
    tab = _make_prep()(e_pad, w5, spt)
    tabq = tab.reshape(MAX_DIST + 1, NQ, 8, VPAD).transpose(
        1, 0, 2, 3).reshape(NQ, (MAX_DIST + 1) * 8 * VPAD)

    eidx = edge_input.reshape(S * MAX_DIST * 3).astype(jnp.int32)
    sp1d = spatial_pos.reshape(S).astype(jnp.int32)
    core = _make_sc()(tabq, eidx, sp1d)

    tt = graph_token_virtual_distance_w.reshape(H, 1)
    return _make_asm()(attn_bias, tt, core)
